# Initial kernel scaffold; baseline (speedup 1.0000x reference)
import functools

import jax
import jax.numpy as jnp
from jax import lax
from jax.experimental import pallas as pl
from jax.experimental.pallas import tpu as pltpu
from jax.experimental.pallas import tpu_sc as plsc

N = 10000
E = 320000
D_IN = 128
H = 64
G = 128

NC = 2
NS = 16
NW = NC * NS

N_PAD = 10016
RPT = N_PAD // NS
E_PAD = 327680
EPT = E_PAD // NW
CHUNK = 128
NCHUNK = EPT // CHUNK

_MESH = plsc.VectorSubcoreMesh(core_axis_name="c", subcore_axis_name="s")


def _sc_agg_body(table, src, dst, zeros, out, src_v, dst_v, rows_v, acc_sh):
    c = lax.axis_index("c")
    s = lax.axis_index("s")
    wid = c * NS + s
    pltpu.sync_copy(zeros.at[pl.ds(s * RPT, RPT)], acc_sh.at[pl.ds(s * RPT, RPT)])
    plsc.subcore_barrier()
    base = wid * EPT

    def body(k, carry):
        off = base + k * CHUNK
        pltpu.sync_copy(src.at[pl.ds(off, CHUNK)], src_v)
        pltpu.sync_copy(dst.at[pl.ds(off, CHUNK)], dst_v)
        pltpu.sync_copy(table.at[src_v], rows_v)
        pltpu.sync_copy(rows_v, acc_sh.at[dst_v], add=True)
        return carry

    lax.fori_loop(0, NCHUNK, body, 0)
    plsc.subcore_barrier()
    pltpu.sync_copy(acc_sh.at[pl.ds(s * RPT, RPT)],
                    out.at[c, pl.ds(s * RPT, RPT)])


_sc_agg = functools.partial(
    pl.kernel,
    out_type=jax.ShapeDtypeStruct((NC, N_PAD, H), jnp.float32),
    mesh=_MESH,
    scratch_types=[
        pltpu.VMEM((CHUNK,), jnp.int32),
        pltpu.VMEM((CHUNK,), jnp.int32),
        pltpu.VMEM((CHUNK, H), jnp.float32),
        pltpu.VMEM_SHARED((N_PAD, H), jnp.float32),
    ],
)(_sc_agg_body)


def _sc_count_body(dst, ones, zeros, out, dst_v, ones_v, acc_sh):
    c = lax.axis_index("c")
    s = lax.axis_index("s")
    wid = c * NS + s
    pltpu.sync_copy(zeros.at[pl.ds(s * RPT, RPT)], acc_sh.at[pl.ds(s * RPT, RPT)])
    pltpu.sync_copy(ones, ones_v)
    plsc.subcore_barrier()
    base = wid * EPT

    def body(k, carry):
        off = base + k * CHUNK
        pltpu.sync_copy(dst.at[pl.ds(off, CHUNK)], dst_v)
        pltpu.sync_copy(ones_v, acc_sh.at[dst_v], add=True)
        return carry

    lax.fori_loop(0, NCHUNK, body, 0)
    plsc.subcore_barrier()
    pltpu.sync_copy(acc_sh.at[pl.ds(s * RPT, RPT)],
                    out.at[c, pl.ds(s * RPT, RPT)])


_sc_count = functools.partial(
    pl.kernel,
    out_type=jax.ShapeDtypeStruct((NC, N_PAD, 16), jnp.float32),
    mesh=_MESH,
    scratch_types=[
        pltpu.VMEM((CHUNK,), jnp.int32),
        pltpu.VMEM((CHUNK, 16), jnp.float32),
        pltpu.VMEM_SHARED((N_PAD, 16), jnp.float32),
    ],
)(_sc_count_body)


def _pre_body(x_ref, wl_ref, wr_ref, hl_ref, hr_ref):
    x = x_ref[...]
    hl_ref[...] = jnp.dot(x, wl_ref[...], preferred_element_type=jnp.float32)
    hr_ref[...] = jnp.dot(x, wr_ref[...], preferred_element_type=jnp.float32)


_tc_pre = pl.pallas_call(
    _pre_body,
    out_shape=(jax.ShapeDtypeStruct((N, H), jnp.float32),
               jax.ShapeDtypeStruct((N, H), jnp.float32)),
)


def _norm_relu(a0, a1, c0, c1, hr, bl, gamma, beta):
    agg = a0 + a1
    cnt = (c0 + c1)[:, 0:1]
    t = agg / jnp.maximum(cnt, 1.0) + bl + hr
    mu = jnp.mean(t, axis=0, keepdims=True)
    var = jnp.mean((t - mu) * (t - mu), axis=0, keepdims=True)
    hn = (t - mu) / jnp.sqrt(var + 1e-5) * gamma + beta
    return jnp.maximum(hn, 0.0)


def _mid_body(a0_ref, a1_ref, c0_ref, c1_ref, hr_ref, bl_ref, g_ref, b_ref,
              wl_ref, wr_ref, hl_o, hr_o):
    h = _norm_relu(a0_ref[...], a1_ref[...], c0_ref[...], c1_ref[...],
                   hr_ref[...], bl_ref[...], g_ref[...], b_ref[...])
    hl_o[...] = jnp.dot(h, wl_ref[...], preferred_element_type=jnp.float32)
    hr_o[...] = jnp.dot(h, wr_ref[...], preferred_element_type=jnp.float32)


_tc_mid = pl.pallas_call(
    _mid_body,
    out_shape=(jax.ShapeDtypeStruct((N, H), jnp.float32),
               jax.ShapeDtypeStruct((N, H), jnp.float32)),
)


def _post_body(a0_ref, a1_ref, c0_ref, c1_ref, hr_ref, bl_ref, g_ref, b_ref,
               batch_ref, gattr_ref, w1a_ref, w1b_ref, b1_ref, w2_ref, b2_ref,
               out_ref):
    h = _norm_relu(a0_ref[...], a1_ref[...], c0_ref[...], c1_ref[...],
                   hr_ref[...], bl_ref[...], g_ref[...], b_ref[...])
    gid = lax.broadcasted_iota(jnp.int32, (G, N), 0)
    mask = (gid == batch_ref[...]).astype(jnp.float32)
    s = jnp.dot(mask, h, preferred_element_type=jnp.float32)
    cntg = jnp.sum(mask, axis=1, keepdims=True)
    emb = s / jnp.maximum(cntg, 1.0)
    z = (jnp.dot(emb, w1a_ref[...], preferred_element_type=jnp.float32)
         + jnp.dot(gattr_ref[...], w1b_ref[...], preferred_element_type=jnp.float32)
         + b1_ref[...])
    z = jnp.maximum(z, 0.0)
    out_ref[...] = (jnp.dot(z, w2_ref[...], preferred_element_type=jnp.float32)
                    + b2_ref[...])


_tc_post = pl.pallas_call(
    _post_body,
    out_shape=jax.ShapeDtypeStruct((G, 2), jnp.float32),
)


def kernel(x, graph_attr, params, edge_index, batch):
    src = edge_index[0]
    dst = edge_index[1]
    pad = E_PAD - E
    src_p = jnp.concatenate([src, jnp.zeros((pad,), jnp.int32)])
    dst_p = jnp.concatenate(
        [dst, N + (jnp.arange(pad, dtype=jnp.int32) % (N_PAD - N))])
    zeros64 = jnp.zeros((N_PAD, H), jnp.float32)
    zeros16 = jnp.zeros((N_PAD, 16), jnp.float32)
    ones16 = jnp.ones((CHUNK, 16), jnp.float32)

    cnt2 = _sc_count(dst_p, ones16, zeros16)
    c0 = cnt2[0, :N]
    c1 = cnt2[1, :N]

    p0 = params['conv0']
    hl, hr = _tc_pre(x, p0['Wl'], p0['Wr'])

    out = None
    for i in range(3):
        p = params['conv%d' % i]
        bl = p['bl'].reshape(1, H)
        gam = p['gamma'].reshape(1, H)
        bet = p['beta'].reshape(1, H)
        acc2 = _sc_agg(hl, src_p, dst_p, zeros64)
        a0 = acc2[0, :N]
        a1 = acc2[1, :N]
        if i < 2:
            pn = params['conv%d' % (i + 1)]
            hl, hr = _tc_mid(a0, a1, c0, c1, hr, bl, gam, bet,
                             pn['Wl'], pn['Wr'])
        else:
            w1 = params['W1']
            out = _tc_post(a0, a1, c0, c1, hr, bl, gam, bet,
                           batch.reshape(1, N), graph_attr,
                           w1[:H], w1[H:], params['b1'].reshape(1, H // 2),
                           params['W2'], params['b2'].reshape(1, 2))
    return out


# speedup vs baseline: 3.9065x; 3.9065x over previous
import functools

import jax
import jax.numpy as jnp
from jax import lax
from jax.experimental import pallas as pl
from jax.experimental.pallas import tpu as pltpu
from jax.experimental.pallas import tpu_sc as plsc

N = 10000
E = 320000
D_IN = 128
H = 64
G = 128

NC = 2
NS = 16
NW = NC * NS

N_PAD = 10112
RPT = N_PAD // NS
E_PAD = 327680
EPT = E_PAD // NW
CHUNK = 128
NCHUNK = EPT // CHUNK

_MESH = plsc.VectorSubcoreMesh(core_axis_name="c", subcore_axis_name="s")
_SC_PARAMS = pltpu.CompilerParams(use_tc_tiling_on_sc=False)


def _sc_agg_body(table, src, dst, zeros, out, src_v, dst_v, rows_v, acc_sh):
    c = lax.axis_index("c")
    s = lax.axis_index("s")
    wid = c * NS + s
    pltpu.sync_copy(zeros.at[pl.ds(s * RPT, RPT)], acc_sh.at[pl.ds(s * RPT, RPT)])
    plsc.subcore_barrier()
    base = wid * EPT

    def body(k, carry):
        off = base + k * CHUNK
        pltpu.sync_copy(src.at[pl.ds(off, CHUNK)], src_v)
        pltpu.sync_copy(dst.at[pl.ds(off, CHUNK)], dst_v)
        pltpu.sync_copy(table.at[src_v], rows_v)
        pltpu.sync_copy(rows_v, acc_sh.at[dst_v], add=True)
        return carry

    lax.fori_loop(0, NCHUNK, body, 0)
    plsc.subcore_barrier()
    pltpu.sync_copy(acc_sh.at[pl.ds(s * RPT, RPT)],
                    out.at[c, pl.ds(s * RPT, RPT)])


_sc_agg = functools.partial(
    pl.kernel,
    out_type=jax.ShapeDtypeStruct((NC, N_PAD, H), jnp.float32),
    mesh=_MESH,
    compiler_params=_SC_PARAMS,
    scratch_types=[
        pltpu.VMEM((CHUNK,), jnp.int32),
        pltpu.VMEM((CHUNK,), jnp.int32),
        pltpu.VMEM((CHUNK, H), jnp.float32),
        pltpu.VMEM_SHARED((N_PAD, H), jnp.float32),
    ],
)(_sc_agg_body)


def _sc_count_body(dst, ones, zeros, out, dst_v, ones_v, acc_sh):
    c = lax.axis_index("c")
    s = lax.axis_index("s")
    wid = c * NS + s
    pltpu.sync_copy(zeros.at[pl.ds(s * RPT, RPT)], acc_sh.at[pl.ds(s * RPT, RPT)])
    pltpu.sync_copy(ones, ones_v)
    plsc.subcore_barrier()
    base = wid * EPT

    def body(k, carry):
        off = base + k * CHUNK
        pltpu.sync_copy(dst.at[pl.ds(off, CHUNK)], dst_v)
        pltpu.sync_copy(ones_v, acc_sh.at[dst_v], add=True)
        return carry

    lax.fori_loop(0, NCHUNK, body, 0)
    plsc.subcore_barrier()
    pltpu.sync_copy(acc_sh.at[pl.ds(s * RPT, RPT)],
                    out.at[c, pl.ds(s * RPT, RPT)])


_sc_count = functools.partial(
    pl.kernel,
    out_type=jax.ShapeDtypeStruct((NC, N_PAD, 16), jnp.float32),
    mesh=_MESH,
    compiler_params=_SC_PARAMS,
    scratch_types=[
        pltpu.VMEM((CHUNK,), jnp.int32),
        pltpu.VMEM((CHUNK, 16), jnp.float32),
        pltpu.VMEM_SHARED((N_PAD, 16), jnp.float32),
    ],
)(_sc_count_body)


def _pre_body(x_ref, wl_ref, wr_ref, hl_ref, hr_ref):
    x = x_ref[...]
    hl_ref[...] = jnp.dot(x, wl_ref[...], preferred_element_type=jnp.float32)
    hr_ref[...] = jnp.dot(x, wr_ref[...], preferred_element_type=jnp.float32)


_tc_pre = pl.pallas_call(
    _pre_body,
    out_shape=(jax.ShapeDtypeStruct((N, H), jnp.float32),
               jax.ShapeDtypeStruct((N, H), jnp.float32)),
)


def _norm_relu(a0, a1, c0, c1, hr, bl, gamma, beta):
    agg = a0 + a1
    cnt = (c0 + c1)[:, 0:1]
    t = agg / jnp.maximum(cnt, 1.0) + bl + hr
    mu = jnp.mean(t, axis=0, keepdims=True)
    var = jnp.mean((t - mu) * (t - mu), axis=0, keepdims=True)
    hn = (t - mu) / jnp.sqrt(var + 1e-5) * gamma + beta
    return jnp.maximum(hn, 0.0)


def _mid_body(a0_ref, a1_ref, c0_ref, c1_ref, hr_ref, bl_ref, g_ref, b_ref,
              wl_ref, wr_ref, hl_o, hr_o):
    h = _norm_relu(a0_ref[...], a1_ref[...], c0_ref[...], c1_ref[...],
                   hr_ref[...], bl_ref[...], g_ref[...], b_ref[...])
    hl_o[...] = jnp.dot(h, wl_ref[...], preferred_element_type=jnp.float32)
    hr_o[...] = jnp.dot(h, wr_ref[...], preferred_element_type=jnp.float32)


_tc_mid = pl.pallas_call(
    _mid_body,
    out_shape=(jax.ShapeDtypeStruct((N, H), jnp.float32),
               jax.ShapeDtypeStruct((N, H), jnp.float32)),
)


def _post_body(a0_ref, a1_ref, c0_ref, c1_ref, hr_ref, bl_ref, g_ref, b_ref,
               batch_ref, gattr_ref, w1a_ref, w1b_ref, b1_ref, w2_ref, b2_ref,
               out_ref):
    h = _norm_relu(a0_ref[...], a1_ref[...], c0_ref[...], c1_ref[...],
                   hr_ref[...], bl_ref[...], g_ref[...], b_ref[...])
    gid = lax.broadcasted_iota(jnp.int32, (G, N), 0)
    mask = (gid == batch_ref[...]).astype(jnp.float32)
    s = jnp.dot(mask, h, preferred_element_type=jnp.float32)
    cntg = jnp.sum(mask, axis=1, keepdims=True)
    emb = s / jnp.maximum(cntg, 1.0)
    z = (jnp.dot(emb, w1a_ref[...], preferred_element_type=jnp.float32)
         + jnp.dot(gattr_ref[...], w1b_ref[...], preferred_element_type=jnp.float32)
         + b1_ref[...])
    z = jnp.maximum(z, 0.0)
    out_ref[...] = (jnp.dot(z, w2_ref[...], preferred_element_type=jnp.float32)
                    + b2_ref[...])


_tc_post = pl.pallas_call(
    _post_body,
    out_shape=jax.ShapeDtypeStruct((G, 2), jnp.float32),
)


def kernel(x, graph_attr, params, edge_index, batch):
    src = edge_index[0]
    dst = edge_index[1]
    pad = E_PAD - E
    src_p = jnp.concatenate([src, jnp.zeros((pad,), jnp.int32)])
    dst_p = jnp.concatenate(
        [dst, N + (jnp.arange(pad, dtype=jnp.int32) % (N_PAD - N))])
    zeros64 = jnp.zeros((N_PAD, H), jnp.float32)
    zeros16 = jnp.zeros((N_PAD, 16), jnp.float32)
    ones16 = jnp.ones((CHUNK, 16), jnp.float32)

    cnt2 = _sc_count(dst_p, ones16, zeros16)
    c0 = cnt2[0, :N]
    c1 = cnt2[1, :N]

    p0 = params['conv0']
    hl, hr = _tc_pre(x, p0['Wl'], p0['Wr'])

    out = None
    for i in range(3):
        p = params['conv%d' % i]
        bl = p['bl'].reshape(1, H)
        gam = p['gamma'].reshape(1, H)
        bet = p['beta'].reshape(1, H)
        acc2 = _sc_agg(hl, src_p, dst_p, zeros64)
        a0 = acc2[0, :N]
        a1 = acc2[1, :N]
        if i < 2:
            pn = params['conv%d' % (i + 1)]
            hl, hr = _tc_mid(a0, a1, c0, c1, hr, bl, gam, bet,
                             pn['Wl'], pn['Wr'])
        else:
            w1 = params['W1']
            out = _tc_post(a0, a1, c0, c1, hr, bl, gam, bet,
                           batch.reshape(1, N), graph_attr,
                           w1[:H], w1[H:], params['b1'].reshape(1, H // 2),
                           params['W2'], params['b2'].reshape(1, 2))
    return out


# speedup vs baseline: 9.5735x; 2.4507x over previous
import functools

import jax
import jax.numpy as jnp
from jax import lax
from jax.experimental import pallas as pl
from jax.experimental.pallas import tpu as pltpu
from jax.experimental.pallas import tpu_sc as plsc

N = 10000
E = 320000
D_IN = 128
H = 64
G = 128

NC = 2
NS = 16
NW = NC * NS

N_PAD = 10112
RPT = N_PAD // NS
EPT = E // NW
CHUNK = 400
NCHUNK = EPT // CHUNK

_MESH = plsc.VectorSubcoreMesh(core_axis_name="c", subcore_axis_name="s")
_SC_PARAMS = pltpu.CompilerParams(use_tc_tiling_on_sc=False)


def _sc_agg_body(table, src, dst, zeros, out, src_v, dst_v, rows_v, acc_sh):
    c = lax.axis_index("c")
    s = lax.axis_index("s")
    wid = c * NS + s
    pltpu.sync_copy(zeros.at[pl.ds(s * RPT, RPT)], acc_sh.at[pl.ds(s * RPT, RPT)])
    plsc.subcore_barrier()
    base = wid * EPT

    def body(k, carry):
        off = base + k * CHUNK
        pltpu.sync_copy(src.at[pl.ds(off, CHUNK)], src_v)
        pltpu.sync_copy(dst.at[pl.ds(off, CHUNK)], dst_v)
        pltpu.sync_copy(table.at[src_v], rows_v)
        pltpu.sync_copy(rows_v, acc_sh.at[dst_v], add=True)
        return carry

    lax.fori_loop(0, NCHUNK, body, 0)
    plsc.subcore_barrier()
    pltpu.sync_copy(acc_sh.at[pl.ds(s * RPT, RPT)],
                    out.at[c, pl.ds(s * RPT, RPT)])


_sc_agg = functools.partial(
    pl.kernel,
    out_type=jax.ShapeDtypeStruct((NC, N_PAD, H), jnp.float32),
    mesh=_MESH,
    compiler_params=_SC_PARAMS,
    scratch_types=[
        pltpu.VMEM((CHUNK,), jnp.int32),
        pltpu.VMEM((CHUNK,), jnp.int32),
        pltpu.VMEM((CHUNK, H), jnp.float32),
        pltpu.VMEM_SHARED((N_PAD, H), jnp.float32),
    ],
)(_sc_agg_body)


def _sc_count_body(dst, ones, zeros, out, dst_v, ones_v, acc_sh):
    c = lax.axis_index("c")
    s = lax.axis_index("s")
    wid = c * NS + s
    pltpu.sync_copy(zeros.at[pl.ds(s * RPT, RPT)], acc_sh.at[pl.ds(s * RPT, RPT)])
    pltpu.sync_copy(ones, ones_v)
    plsc.subcore_barrier()
    base = wid * EPT

    def body(k, carry):
        off = base + k * CHUNK
        pltpu.sync_copy(dst.at[pl.ds(off, CHUNK)], dst_v)
        pltpu.sync_copy(ones_v, acc_sh.at[dst_v], add=True)
        return carry

    lax.fori_loop(0, NCHUNK, body, 0)
    plsc.subcore_barrier()
    pltpu.sync_copy(acc_sh.at[pl.ds(s * RPT, RPT)],
                    out.at[c, pl.ds(s * RPT, RPT)])


_sc_count = functools.partial(
    pl.kernel,
    out_type=jax.ShapeDtypeStruct((NC, N_PAD, 16), jnp.float32),
    mesh=_MESH,
    compiler_params=_SC_PARAMS,
    scratch_types=[
        pltpu.VMEM((CHUNK,), jnp.int32),
        pltpu.VMEM((CHUNK, 16), jnp.float32),
        pltpu.VMEM_SHARED((N_PAD, 16), jnp.float32),
    ],
)(_sc_count_body)


def _pre_body(x_ref, wl_ref, wr_ref, hl_ref, hr_ref):
    x = x_ref[...]
    hl_ref[...] = jnp.dot(x, wl_ref[...], preferred_element_type=jnp.float32)
    hr_ref[...] = jnp.dot(x, wr_ref[...], preferred_element_type=jnp.float32)


_tc_pre = pl.pallas_call(
    _pre_body,
    out_shape=(jax.ShapeDtypeStruct((N, H), jnp.float32),
               jax.ShapeDtypeStruct((N, H), jnp.float32)),
)


def _norm_relu(a0, a1, c0, c1, hr, bl, gamma, beta):
    agg = a0 + a1
    cnt = (c0 + c1)[:, 0:1]
    t = agg / jnp.maximum(cnt, 1.0) + bl + hr
    mu = jnp.mean(t, axis=0, keepdims=True)
    var = jnp.mean((t - mu) * (t - mu), axis=0, keepdims=True)
    hn = (t - mu) / jnp.sqrt(var + 1e-5) * gamma + beta
    return jnp.maximum(hn, 0.0)


def _mid_body(a0_ref, a1_ref, c0_ref, c1_ref, hr_ref, bl_ref, g_ref, b_ref,
              wl_ref, wr_ref, hl_o, hr_o):
    h = _norm_relu(a0_ref[...], a1_ref[...], c0_ref[...], c1_ref[...],
                   hr_ref[...], bl_ref[...], g_ref[...], b_ref[...])
    hl_o[...] = jnp.dot(h, wl_ref[...], preferred_element_type=jnp.float32)
    hr_o[...] = jnp.dot(h, wr_ref[...], preferred_element_type=jnp.float32)


_tc_mid = pl.pallas_call(
    _mid_body,
    out_shape=(jax.ShapeDtypeStruct((N, H), jnp.float32),
               jax.ShapeDtypeStruct((N, H), jnp.float32)),
)


def _post_body(a0_ref, a1_ref, c0_ref, c1_ref, hr_ref, bl_ref, g_ref, b_ref,
               batch_ref, gattr_ref, w1a_ref, w1b_ref, b1_ref, w2_ref, b2_ref,
               out_ref):
    h = _norm_relu(a0_ref[...], a1_ref[...], c0_ref[...], c1_ref[...],
                   hr_ref[...], bl_ref[...], g_ref[...], b_ref[...])
    gid = lax.broadcasted_iota(jnp.int32, (G, N), 0)
    mask = (gid == batch_ref[...]).astype(jnp.float32)
    s = jnp.dot(mask, h, preferred_element_type=jnp.float32)
    cntg = jnp.sum(mask, axis=1, keepdims=True)
    emb = s / jnp.maximum(cntg, 1.0)
    z = (jnp.dot(emb, w1a_ref[...], preferred_element_type=jnp.float32)
         + jnp.dot(gattr_ref[...], w1b_ref[...], preferred_element_type=jnp.float32)
         + b1_ref[...])
    z = jnp.maximum(z, 0.0)
    out_ref[...] = (jnp.dot(z, w2_ref[...], preferred_element_type=jnp.float32)
                    + b2_ref[...])


_tc_post = pl.pallas_call(
    _post_body,
    out_shape=jax.ShapeDtypeStruct((G, 2), jnp.float32),
)


def kernel(x, graph_attr, params, edge_index, batch):
    src_p = edge_index[0]
    dst_p = edge_index[1]
    zeros64 = jnp.zeros((N_PAD, H), jnp.float32)
    zeros16 = jnp.zeros((N_PAD, 16), jnp.float32)
    ones16 = jnp.ones((CHUNK, 16), jnp.float32)

    cnt2 = _sc_count(dst_p, ones16, zeros16)
    c0 = cnt2[0, :N]
    c1 = cnt2[1, :N]

    p0 = params['conv0']
    hl, hr = _tc_pre(x, p0['Wl'], p0['Wr'])

    out = None
    for i in range(3):
        p = params['conv%d' % i]
        bl = p['bl'].reshape(1, H)
        gam = p['gamma'].reshape(1, H)
        bet = p['beta'].reshape(1, H)
        acc2 = _sc_agg(hl, src_p, dst_p, zeros64)
        a0 = acc2[0, :N]
        a1 = acc2[1, :N]
        if i < 2:
            pn = params['conv%d' % (i + 1)]
            hl, hr = _tc_mid(a0, a1, c0, c1, hr, bl, gam, bet,
                             pn['Wl'], pn['Wr'])
        else:
            w1 = params['W1']
            out = _tc_post(a0, a1, c0, c1, hr, bl, gam, bet,
                           batch.reshape(1, N), graph_attr,
                           w1[:H], w1[H:], params['b1'].reshape(1, H // 2),
                           params['W2'], params['b2'].reshape(1, 2))
    return out


# speedup vs baseline: 10.9017x; 1.1387x over previous
import functools

import jax
import jax.numpy as jnp
from jax import lax
from jax.experimental import pallas as pl
from jax.experimental.pallas import tpu as pltpu
from jax.experimental.pallas import tpu_sc as plsc

N = 10000
E = 320000
D_IN = 128
H = 64
G = 128

NC = 2
NS = 16
NW = NC * NS

N_PAD = 10112
RPT = N_PAD // NS
EPT = E // NW
CHUNK = 1000
NCHUNK = EPT // CHUNK

_MESH = plsc.VectorSubcoreMesh(core_axis_name="c", subcore_axis_name="s")
_SC_PARAMS = pltpu.CompilerParams(use_tc_tiling_on_sc=False)


def _sc_agg_body(table, src, dst, zeros, out, src_v, dst_v, rows_v, acc_sh):
    c = lax.axis_index("c")
    s = lax.axis_index("s")
    wid = c * NS + s
    pltpu.sync_copy(zeros.at[pl.ds(s * RPT, RPT)], acc_sh.at[pl.ds(s * RPT, RPT)])
    plsc.subcore_barrier()
    base = wid * EPT

    def body(k, carry):
        off = base + k * CHUNK
        pltpu.sync_copy(src.at[pl.ds(off, CHUNK)], src_v)
        pltpu.sync_copy(dst.at[pl.ds(off, CHUNK)], dst_v)
        pltpu.sync_copy(table.at[src_v], rows_v)
        pltpu.sync_copy(rows_v, acc_sh.at[dst_v], add=True)
        return carry

    lax.fori_loop(0, NCHUNK, body, 0)
    plsc.subcore_barrier()
    pltpu.sync_copy(acc_sh.at[pl.ds(s * RPT, RPT)],
                    out.at[c, pl.ds(s * RPT, RPT)])


_sc_agg = functools.partial(
    pl.kernel,
    out_type=jax.ShapeDtypeStruct((NC, N_PAD, H), jnp.float32),
    mesh=_MESH,
    compiler_params=_SC_PARAMS,
    scratch_types=[
        pltpu.VMEM((CHUNK,), jnp.int32),
        pltpu.VMEM((CHUNK,), jnp.int32),
        pltpu.VMEM((CHUNK, H), jnp.float32),
        pltpu.VMEM_SHARED((N_PAD, H), jnp.float32),
    ],
)(_sc_agg_body)


def _sc_count_body(dst, ones, zeros, out, dst_v, ones_v, acc_sh):
    c = lax.axis_index("c")
    s = lax.axis_index("s")
    wid = c * NS + s
    pltpu.sync_copy(zeros.at[pl.ds(s * RPT, RPT)], acc_sh.at[pl.ds(s * RPT, RPT)])
    pltpu.sync_copy(ones, ones_v)
    plsc.subcore_barrier()
    base = wid * EPT

    def body(k, carry):
        off = base + k * CHUNK
        pltpu.sync_copy(dst.at[pl.ds(off, CHUNK)], dst_v)
        pltpu.sync_copy(ones_v, acc_sh.at[dst_v], add=True)
        return carry

    lax.fori_loop(0, NCHUNK, body, 0)
    plsc.subcore_barrier()
    pltpu.sync_copy(acc_sh.at[pl.ds(s * RPT, RPT)],
                    out.at[c, pl.ds(s * RPT, RPT)])


_sc_count = functools.partial(
    pl.kernel,
    out_type=jax.ShapeDtypeStruct((NC, N_PAD, 16), jnp.float32),
    mesh=_MESH,
    compiler_params=_SC_PARAMS,
    scratch_types=[
        pltpu.VMEM((CHUNK,), jnp.int32),
        pltpu.VMEM((CHUNK, 16), jnp.float32),
        pltpu.VMEM_SHARED((N_PAD, 16), jnp.float32),
    ],
)(_sc_count_body)


def _pre_body(x_ref, wl_ref, wr_ref, hl_ref, hr_ref):
    x = x_ref[...]
    hl_ref[...] = jnp.dot(x, wl_ref[...], preferred_element_type=jnp.float32)
    hr_ref[...] = jnp.dot(x, wr_ref[...], preferred_element_type=jnp.float32)


_tc_pre = pl.pallas_call(
    _pre_body,
    out_shape=(jax.ShapeDtypeStruct((N, H), jnp.float32),
               jax.ShapeDtypeStruct((N, H), jnp.float32)),
)


def _norm_relu(a0, a1, c0, c1, hr, bl, gamma, beta):
    agg = a0 + a1
    cnt = (c0 + c1)[:, 0:1]
    t = agg / jnp.maximum(cnt, 1.0) + bl + hr
    mu = jnp.mean(t, axis=0, keepdims=True)
    var = jnp.mean((t - mu) * (t - mu), axis=0, keepdims=True)
    hn = (t - mu) / jnp.sqrt(var + 1e-5) * gamma + beta
    return jnp.maximum(hn, 0.0)


def _mid_body(a0_ref, a1_ref, c0_ref, c1_ref, hr_ref, bl_ref, g_ref, b_ref,
              wl_ref, wr_ref, hl_o, hr_o):
    h = _norm_relu(a0_ref[...], a1_ref[...], c0_ref[...], c1_ref[...],
                   hr_ref[...], bl_ref[...], g_ref[...], b_ref[...])
    hl_o[...] = jnp.dot(h, wl_ref[...], preferred_element_type=jnp.float32)
    hr_o[...] = jnp.dot(h, wr_ref[...], preferred_element_type=jnp.float32)


_tc_mid = pl.pallas_call(
    _mid_body,
    out_shape=(jax.ShapeDtypeStruct((N, H), jnp.float32),
               jax.ShapeDtypeStruct((N, H), jnp.float32)),
)


def _post_body(a0_ref, a1_ref, c0_ref, c1_ref, hr_ref, bl_ref, g_ref, b_ref,
               batch_ref, gattr_ref, w1a_ref, w1b_ref, b1_ref, w2_ref, b2_ref,
               out_ref):
    h = _norm_relu(a0_ref[...], a1_ref[...], c0_ref[...], c1_ref[...],
                   hr_ref[...], bl_ref[...], g_ref[...], b_ref[...])
    gid = lax.broadcasted_iota(jnp.int32, (G, N), 0)
    mask = (gid == batch_ref[...]).astype(jnp.float32)
    s = jnp.dot(mask, h, preferred_element_type=jnp.float32)
    cntg = jnp.sum(mask, axis=1, keepdims=True)
    emb = s / jnp.maximum(cntg, 1.0)
    z = (jnp.dot(emb, w1a_ref[...], preferred_element_type=jnp.float32)
         + jnp.dot(gattr_ref[...], w1b_ref[...], preferred_element_type=jnp.float32)
         + b1_ref[...])
    z = jnp.maximum(z, 0.0)
    out_ref[...] = (jnp.dot(z, w2_ref[...], preferred_element_type=jnp.float32)
                    + b2_ref[...])


_tc_post = pl.pallas_call(
    _post_body,
    out_shape=jax.ShapeDtypeStruct((G, 2), jnp.float32),
)


def kernel(x, graph_attr, params, edge_index, batch):
    src_p = edge_index[0]
    dst_p = edge_index[1]
    zeros64 = jnp.zeros((N_PAD, H), jnp.float32)
    zeros16 = jnp.zeros((N_PAD, 16), jnp.float32)
    ones16 = jnp.ones((CHUNK, 16), jnp.float32)

    cnt2 = _sc_count(dst_p, ones16, zeros16)
    c0 = cnt2[0, :N]
    c1 = cnt2[1, :N]

    p0 = params['conv0']
    hl, hr = _tc_pre(x, p0['Wl'], p0['Wr'])

    out = None
    for i in range(3):
        p = params['conv%d' % i]
        bl = p['bl'].reshape(1, H)
        gam = p['gamma'].reshape(1, H)
        bet = p['beta'].reshape(1, H)
        acc2 = _sc_agg(hl, src_p, dst_p, zeros64)
        a0 = acc2[0, :N]
        a1 = acc2[1, :N]
        if i < 2:
            pn = params['conv%d' % (i + 1)]
            hl, hr = _tc_mid(a0, a1, c0, c1, hr, bl, gam, bet,
                             pn['Wl'], pn['Wr'])
        else:
            w1 = params['W1']
            out = _tc_post(a0, a1, c0, c1, hr, bl, gam, bet,
                           batch.reshape(1, N), graph_attr,
                           w1[:H], w1[H:], params['b1'].reshape(1, H // 2),
                           params['W2'], params['b2'].reshape(1, 2))
    return out


# speedup vs baseline: 13.5149x; 1.2397x over previous
import functools

import jax
import jax.numpy as jnp
from jax import lax
from jax.experimental import pallas as pl
from jax.experimental.pallas import tpu as pltpu
from jax.experimental.pallas import tpu_sc as plsc

N = 10000
E = 320000
D_IN = 128
H = 64
G = 128

NC = 2
NS = 16
NW = NC * NS

N_PAD = 10112
RPT = N_PAD // NS
EPT = E // NW
CHUNK = 400
NCHUNK = EPT // CHUNK

_MESH = plsc.VectorSubcoreMesh(core_axis_name="c", subcore_axis_name="s")
_SC_PARAMS = pltpu.CompilerParams(use_tc_tiling_on_sc=False)


def _sc_agg_body(table, src, dst, zeros, out, srcall, dstall, rows_v, acc_sh,
                 zsem, gsem, ssem):
    c = lax.axis_index("c")
    s = lax.axis_index("s")
    wid = c * NS + s
    base = wid * EPT
    zd = pltpu.async_copy(zeros.at[pl.ds(s * RPT, RPT)],
                          acc_sh.at[pl.ds(s * RPT, RPT)], zsem)
    pltpu.sync_copy(src.at[pl.ds(base, EPT)], srcall)
    pltpu.sync_copy(dst.at[pl.ds(base, EPT)], dstall)

    def gather(k, b):
        return pltpu.async_copy(
            table.at[srcall.at[pl.ds(k * CHUNK, CHUNK)]], rows_v.at[b], gsem)

    g = {0: gather(0, 0)}
    zd.wait()
    plsc.subcore_barrier()
    sd = {}
    for k in range(NCHUNK):
        b = k & 1
        g[k].wait()
        sd[k] = pltpu.async_copy(
            rows_v.at[b], acc_sh.at[dstall.at[pl.ds(k * CHUNK, CHUNK)]],
            ssem, add=True)
        if k >= 1:
            sd[k - 1].wait()
        if k + 1 < NCHUNK:
            g[k + 1] = gather(k + 1, b ^ 1)
    sd[NCHUNK - 1].wait()
    plsc.subcore_barrier()
    pltpu.sync_copy(acc_sh.at[pl.ds(s * RPT, RPT)],
                    out.at[c, pl.ds(s * RPT, RPT)])


_sc_agg = functools.partial(
    pl.kernel,
    out_type=jax.ShapeDtypeStruct((NC, N_PAD, H), jnp.float32),
    mesh=_MESH,
    compiler_params=_SC_PARAMS,
    scratch_types=[
        pltpu.VMEM((EPT,), jnp.int32),
        pltpu.VMEM((EPT,), jnp.int32),
        pltpu.VMEM((2, CHUNK, H), jnp.float32),
        pltpu.VMEM_SHARED((N_PAD, H), jnp.float32),
        pltpu.SemaphoreType.DMA,
        pltpu.SemaphoreType.DMA,
        pltpu.SemaphoreType.DMA,
    ],
)(_sc_agg_body)


def _sc_count_body(dst, ones, zeros, out, dstall, ones_v, acc_sh, zsem, ssem):
    c = lax.axis_index("c")
    s = lax.axis_index("s")
    wid = c * NS + s
    base = wid * EPT
    zd = pltpu.async_copy(zeros.at[pl.ds(s * RPT, RPT)],
                          acc_sh.at[pl.ds(s * RPT, RPT)], zsem)
    pltpu.sync_copy(dst.at[pl.ds(base, EPT)], dstall)
    pltpu.sync_copy(ones, ones_v)
    zd.wait()
    plsc.subcore_barrier()
    sd = []
    for k in range(NCHUNK):
        sd.append(pltpu.async_copy(
            ones_v, acc_sh.at[dstall.at[pl.ds(k * CHUNK, CHUNK)]],
            ssem, add=True))
    for d in sd:
        d.wait()
    plsc.subcore_barrier()
    pltpu.sync_copy(acc_sh.at[pl.ds(s * RPT, RPT)],
                    out.at[c, pl.ds(s * RPT, RPT)])


_sc_count = functools.partial(
    pl.kernel,
    out_type=jax.ShapeDtypeStruct((NC, N_PAD, 16), jnp.float32),
    mesh=_MESH,
    compiler_params=_SC_PARAMS,
    scratch_types=[
        pltpu.VMEM((EPT,), jnp.int32),
        pltpu.VMEM((CHUNK, 16), jnp.float32),
        pltpu.VMEM_SHARED((N_PAD, 16), jnp.float32),
        pltpu.SemaphoreType.DMA,
        pltpu.SemaphoreType.DMA,
    ],
)(_sc_count_body)


def _pre_body(x_ref, wl_ref, wr_ref, hl_ref, hr_ref):
    x = x_ref[...]
    hl_ref[...] = jnp.dot(x, wl_ref[...], preferred_element_type=jnp.float32)
    hr_ref[...] = jnp.dot(x, wr_ref[...], preferred_element_type=jnp.float32)


_tc_pre = pl.pallas_call(
    _pre_body,
    out_shape=(jax.ShapeDtypeStruct((N, H), jnp.float32),
               jax.ShapeDtypeStruct((N, H), jnp.float32)),
)


def _norm_relu(a0, a1, c0, c1, hr, bl, gamma, beta):
    agg = a0 + a1
    cnt = (c0 + c1)[:, 0:1]
    t = agg / jnp.maximum(cnt, 1.0) + bl + hr
    mu = jnp.mean(t, axis=0, keepdims=True)
    var = jnp.mean((t - mu) * (t - mu), axis=0, keepdims=True)
    hn = (t - mu) / jnp.sqrt(var + 1e-5) * gamma + beta
    return jnp.maximum(hn, 0.0)


def _mid_body(a0_ref, a1_ref, c0_ref, c1_ref, hr_ref, bl_ref, g_ref, b_ref,
              wl_ref, wr_ref, hl_o, hr_o):
    h = _norm_relu(a0_ref[...], a1_ref[...], c0_ref[...], c1_ref[...],
                   hr_ref[...], bl_ref[...], g_ref[...], b_ref[...])
    hl_o[...] = jnp.dot(h, wl_ref[...], preferred_element_type=jnp.float32)
    hr_o[...] = jnp.dot(h, wr_ref[...], preferred_element_type=jnp.float32)


_tc_mid = pl.pallas_call(
    _mid_body,
    out_shape=(jax.ShapeDtypeStruct((N, H), jnp.float32),
               jax.ShapeDtypeStruct((N, H), jnp.float32)),
)


def _post_body(a0_ref, a1_ref, c0_ref, c1_ref, hr_ref, bl_ref, g_ref, b_ref,
               batch_ref, gattr_ref, w1a_ref, w1b_ref, b1_ref, w2_ref, b2_ref,
               out_ref):
    h = _norm_relu(a0_ref[...], a1_ref[...], c0_ref[...], c1_ref[...],
                   hr_ref[...], bl_ref[...], g_ref[...], b_ref[...])
    gid = lax.broadcasted_iota(jnp.int32, (G, N), 0)
    mask = (gid == batch_ref[...]).astype(jnp.float32)
    s = jnp.dot(mask, h, preferred_element_type=jnp.float32)
    cntg = jnp.sum(mask, axis=1, keepdims=True)
    emb = s / jnp.maximum(cntg, 1.0)
    z = (jnp.dot(emb, w1a_ref[...], preferred_element_type=jnp.float32)
         + jnp.dot(gattr_ref[...], w1b_ref[...], preferred_element_type=jnp.float32)
         + b1_ref[...])
    z = jnp.maximum(z, 0.0)
    out_ref[...] = (jnp.dot(z, w2_ref[...], preferred_element_type=jnp.float32)
                    + b2_ref[...])


_tc_post = pl.pallas_call(
    _post_body,
    out_shape=jax.ShapeDtypeStruct((G, 2), jnp.float32),
)


def kernel(x, graph_attr, params, edge_index, batch):
    src_p = edge_index[0]
    dst_p = edge_index[1]
    zeros64 = jnp.zeros((N_PAD, H), jnp.float32)
    zeros16 = jnp.zeros((N_PAD, 16), jnp.float32)
    ones16 = jnp.ones((CHUNK, 16), jnp.float32)

    cnt2 = _sc_count(dst_p, ones16, zeros16)
    c0 = cnt2[0, :N]
    c1 = cnt2[1, :N]

    p0 = params['conv0']
    hl, hr = _tc_pre(x, p0['Wl'], p0['Wr'])

    out = None
    for i in range(3):
        p = params['conv%d' % i]
        bl = p['bl'].reshape(1, H)
        gam = p['gamma'].reshape(1, H)
        bet = p['beta'].reshape(1, H)
        acc2 = _sc_agg(hl, src_p, dst_p, zeros64)
        a0 = acc2[0, :N]
        a1 = acc2[1, :N]
        if i < 2:
            pn = params['conv%d' % (i + 1)]
            hl, hr = _tc_mid(a0, a1, c0, c1, hr, bl, gam, bet,
                             pn['Wl'], pn['Wr'])
        else:
            w1 = params['W1']
            out = _tc_post(a0, a1, c0, c1, hr, bl, gam, bet,
                           batch.reshape(1, N), graph_attr,
                           w1[:H], w1[H:], params['b1'].reshape(1, H // 2),
                           params['W2'], params['b2'].reshape(1, 2))
    return out


# speedup vs baseline: 14.6191x; 1.0817x over previous
import functools

import jax
import jax.numpy as jnp
from jax import lax
from jax.experimental import pallas as pl
from jax.experimental.pallas import tpu as pltpu
from jax.experimental.pallas import tpu_sc as plsc

N = 10000
E = 320000
D_IN = 128
H = 64
G = 128

NC = 2
NS = 16
NW = NC * NS

N_PAD = 10112
RPT = N_PAD // NS
EPT = E // NW
CHUNK = 400
NCHUNK = EPT // CHUNK

_MESH = plsc.VectorSubcoreMesh(core_axis_name="c", subcore_axis_name="s")
_SC_PARAMS = pltpu.CompilerParams(use_tc_tiling_on_sc=False)


def _agg_pipeline(table, srcall, dstall, rows_v, acc_sh, gsem, ssem,
                  per_chunk=None):
    def gather(k, b):
        return pltpu.async_copy(
            table.at[srcall.at[pl.ds(k * CHUNK, CHUNK)]], rows_v.at[b], gsem)

    g = {0: gather(0, 0)}
    sd = {}
    for k in range(NCHUNK):
        b = k & 1
        g[k].wait()
        sd[k] = pltpu.async_copy(
            rows_v.at[b], acc_sh.at[dstall.at[pl.ds(k * CHUNK, CHUNK)]],
            ssem, add=True)
        if per_chunk is not None:
            per_chunk(k)
        if k >= 1:
            sd[k - 1].wait()
        if k + 1 < NCHUNK:
            g[k + 1] = gather(k + 1, b ^ 1)
    sd[NCHUNK - 1].wait()


def _sc_agg0_body(table, src, dst, zeros, zeros16, ones, out, cnt_out,
                  srcall, dstall, rows_v, ones_v, acc_sh, cnt_sh,
                  zsem, gsem, ssem, csem):
    c = lax.axis_index("c")
    s = lax.axis_index("s")
    wid = c * NS + s
    base = wid * EPT
    zd = pltpu.async_copy(zeros.at[pl.ds(s * RPT, RPT)],
                          acc_sh.at[pl.ds(s * RPT, RPT)], zsem)
    zd16 = pltpu.async_copy(zeros16.at[pl.ds(s * RPT, RPT)],
                            cnt_sh.at[pl.ds(s * RPT, RPT)], zsem)
    pltpu.sync_copy(src.at[pl.ds(base, EPT)], srcall)
    pltpu.sync_copy(dst.at[pl.ds(base, EPT)], dstall)
    pltpu.sync_copy(ones, ones_v)
    zd.wait()
    zd16.wait()
    plsc.subcore_barrier()

    cd = {}

    def count_scatter(k):
        cd[k] = pltpu.async_copy(
            ones_v, cnt_sh.at[dstall.at[pl.ds(k * CHUNK, CHUNK)]],
            csem, add=True)
        if k >= 2:
            cd[k - 2].wait()

    _agg_pipeline(table, srcall, dstall, rows_v, acc_sh, gsem, ssem,
                  per_chunk=count_scatter)
    cd[NCHUNK - 2].wait()
    cd[NCHUNK - 1].wait()
    plsc.subcore_barrier()
    pltpu.sync_copy(acc_sh.at[pl.ds(s * RPT, RPT)],
                    out.at[c, pl.ds(s * RPT, RPT)])
    pltpu.sync_copy(cnt_sh.at[pl.ds(s * RPT, RPT)],
                    cnt_out.at[c, pl.ds(s * RPT, RPT)])


_sc_agg0 = functools.partial(
    pl.kernel,
    out_type=(jax.ShapeDtypeStruct((NC, N_PAD, H), jnp.float32),
              jax.ShapeDtypeStruct((NC, N_PAD, 16), jnp.float32)),
    mesh=_MESH,
    compiler_params=_SC_PARAMS,
    scratch_types=[
        pltpu.VMEM((EPT,), jnp.int32),
        pltpu.VMEM((EPT,), jnp.int32),
        pltpu.VMEM((2, CHUNK, H), jnp.float32),
        pltpu.VMEM((CHUNK, 16), jnp.float32),
        pltpu.VMEM_SHARED((N_PAD, H), jnp.float32),
        pltpu.VMEM_SHARED((N_PAD, 16), jnp.float32),
        pltpu.SemaphoreType.DMA,
        pltpu.SemaphoreType.DMA,
        pltpu.SemaphoreType.DMA,
        pltpu.SemaphoreType.DMA,
    ],
)(_sc_agg0_body)


def _sc_agg_body(table, src, dst, zeros, out, srcall, dstall, rows_v, acc_sh,
                 zsem, gsem, ssem):
    c = lax.axis_index("c")
    s = lax.axis_index("s")
    wid = c * NS + s
    base = wid * EPT
    zd = pltpu.async_copy(zeros.at[pl.ds(s * RPT, RPT)],
                          acc_sh.at[pl.ds(s * RPT, RPT)], zsem)
    pltpu.sync_copy(src.at[pl.ds(base, EPT)], srcall)
    pltpu.sync_copy(dst.at[pl.ds(base, EPT)], dstall)
    zd.wait()
    plsc.subcore_barrier()
    _agg_pipeline(table, srcall, dstall, rows_v, acc_sh, gsem, ssem)
    plsc.subcore_barrier()
    pltpu.sync_copy(acc_sh.at[pl.ds(s * RPT, RPT)],
                    out.at[c, pl.ds(s * RPT, RPT)])


_sc_agg = functools.partial(
    pl.kernel,
    out_type=jax.ShapeDtypeStruct((NC, N_PAD, H), jnp.float32),
    mesh=_MESH,
    compiler_params=_SC_PARAMS,
    scratch_types=[
        pltpu.VMEM((EPT,), jnp.int32),
        pltpu.VMEM((EPT,), jnp.int32),
        pltpu.VMEM((2, CHUNK, H), jnp.float32),
        pltpu.VMEM_SHARED((N_PAD, H), jnp.float32),
        pltpu.SemaphoreType.DMA,
        pltpu.SemaphoreType.DMA,
        pltpu.SemaphoreType.DMA,
    ],
)(_sc_agg_body)


def _pre_body(x_ref, wl_ref, wr_ref, hl_ref, hr_ref):
    x = x_ref[...]
    hl_ref[...] = jnp.dot(x, wl_ref[...], preferred_element_type=jnp.float32)
    hr_ref[...] = jnp.dot(x, wr_ref[...], preferred_element_type=jnp.float32)


_tc_pre = pl.pallas_call(
    _pre_body,
    out_shape=(jax.ShapeDtypeStruct((N, H), jnp.float32),
               jax.ShapeDtypeStruct((N, H), jnp.float32)),
)


def _norm_relu(acc_ref, cnt_ref, hr_ref, bl_ref, g_ref, b_ref):
    agg = acc_ref[0, :N, :] + acc_ref[1, :N, :]
    cnt = cnt_ref[0, :N, 0:1] + cnt_ref[1, :N, 0:1]
    t = agg / jnp.maximum(cnt, 1.0) + bl_ref[...] + hr_ref[...]
    mu = jnp.mean(t, axis=0, keepdims=True)
    var = jnp.mean((t - mu) * (t - mu), axis=0, keepdims=True)
    hn = (t - mu) / jnp.sqrt(var + 1e-5) * g_ref[...] + b_ref[...]
    return jnp.maximum(hn, 0.0)


def _mid_body(acc_ref, cnt_ref, hr_ref, bl_ref, g_ref, b_ref,
              wl_ref, wr_ref, hl_o, hr_o):
    h = _norm_relu(acc_ref, cnt_ref, hr_ref, bl_ref, g_ref, b_ref)
    hl_o[...] = jnp.dot(h, wl_ref[...], preferred_element_type=jnp.float32)
    hr_o[...] = jnp.dot(h, wr_ref[...], preferred_element_type=jnp.float32)


_tc_mid = pl.pallas_call(
    _mid_body,
    out_shape=(jax.ShapeDtypeStruct((N, H), jnp.float32),
               jax.ShapeDtypeStruct((N, H), jnp.float32)),
)


def _post_body(acc_ref, cnt_ref, hr_ref, bl_ref, g_ref, b_ref,
               batch_ref, gattr_ref, w1_ref, b1_ref, w2_ref, b2_ref,
               out_ref):
    h = _norm_relu(acc_ref, cnt_ref, hr_ref, bl_ref, g_ref, b_ref)
    gid = lax.broadcasted_iota(jnp.int32, (G, N), 0)
    mask = (gid == batch_ref[...]).astype(jnp.float32)
    s = jnp.dot(mask, h, preferred_element_type=jnp.float32)
    cntg = jnp.sum(mask, axis=1, keepdims=True)
    emb = s / jnp.maximum(cntg, 1.0)
    w1 = w1_ref[...]
    z = (jnp.dot(emb, w1[:H], preferred_element_type=jnp.float32)
         + jnp.dot(gattr_ref[...], w1[H:], preferred_element_type=jnp.float32)
         + b1_ref[...])
    z = jnp.maximum(z, 0.0)
    out_ref[...] = (jnp.dot(z, w2_ref[...], preferred_element_type=jnp.float32)
                    + b2_ref[...])


_tc_post = pl.pallas_call(
    _post_body,
    out_shape=jax.ShapeDtypeStruct((G, 2), jnp.float32),
)


def kernel(x, graph_attr, params, edge_index, batch):
    src_p = edge_index[0]
    dst_p = edge_index[1]
    zeros64 = jnp.zeros((N_PAD, H), jnp.float32)
    zeros16 = jnp.zeros((N_PAD, 16), jnp.float32)
    ones16 = jnp.ones((CHUNK, 16), jnp.float32)

    p0 = params['conv0']
    hl, hr = _tc_pre(x, p0['Wl'], p0['Wr'])

    out = None
    cnt2 = None
    for i in range(3):
        p = params['conv%d' % i]
        if i == 0:
            acc2, cnt2 = _sc_agg0(hl, src_p, dst_p, zeros64, zeros16, ones16)
        else:
            acc2 = _sc_agg(hl, src_p, dst_p, zeros64)
        if i < 2:
            pn = params['conv%d' % (i + 1)]
            hl, hr = _tc_mid(acc2, cnt2, hr, p['bl'], p['gamma'], p['beta'],
                             pn['Wl'], pn['Wr'])
        else:
            out = _tc_post(acc2, cnt2, hr, p['bl'], p['gamma'], p['beta'],
                           batch, graph_attr,
                           params['W1'], params['b1'],
                           params['W2'], params['b2'])
    return out


# speedup vs baseline: 15.0953x; 1.0326x over previous
import functools

import jax
import jax.numpy as jnp
from jax import lax
from jax.experimental import pallas as pl
from jax.experimental.pallas import tpu as pltpu
from jax.experimental.pallas import tpu_sc as plsc

N = 10000
E = 320000
D_IN = 128
H = 64
G = 128

NC = 2
NS = 16
NW = NC * NS

N_PAD = 10112
RPT = N_PAD // NS
EPT = E // NW
CHUNK = 400
NCHUNK = EPT // CHUNK

_MESH = plsc.VectorSubcoreMesh(core_axis_name="c", subcore_axis_name="s")
_SC_PARAMS = pltpu.CompilerParams(use_tc_tiling_on_sc=False)


def _agg_pipeline(table, srcall, dstall, rows_v, acc_sh, gsem, ssem,
                  per_chunk=None):
    def gather(k, b):
        return pltpu.async_copy(
            table.at[srcall.at[pl.ds(k * CHUNK, CHUNK)]], rows_v.at[b], gsem)

    g = {0: gather(0, 0)}
    sd = {}
    for k in range(NCHUNK):
        b = k & 1
        g[k].wait()
        sd[k] = pltpu.async_copy(
            rows_v.at[b], acc_sh.at[dstall.at[pl.ds(k * CHUNK, CHUNK)]],
            ssem, add=True)
        if per_chunk is not None:
            per_chunk(k)
        if k >= 1:
            sd[k - 1].wait()
        if k + 1 < NCHUNK:
            g[k + 1] = gather(k + 1, b ^ 1)
    sd[NCHUNK - 1].wait()


def _sc_agg0_body(table, edges, zeros, zeros16, ones, out, cnt_out,
                  srcall, dstall, rows_v, ones_v, acc_sh, cnt_sh,
                  zsem, gsem, ssem, csem):
    c = lax.axis_index("c")
    s = lax.axis_index("s")
    wid = c * NS + s
    base = wid * EPT
    zd = pltpu.async_copy(zeros.at[pl.ds(s * RPT, RPT)],
                          acc_sh.at[pl.ds(s * RPT, RPT)], zsem)
    zd16 = pltpu.async_copy(zeros16.at[pl.ds(s * RPT, RPT)],
                            cnt_sh.at[pl.ds(s * RPT, RPT)], zsem)
    pltpu.sync_copy(edges.at[0, pl.ds(base, EPT)], srcall)
    pltpu.sync_copy(edges.at[1, pl.ds(base, EPT)], dstall)
    pltpu.sync_copy(ones, ones_v)
    zd.wait()
    zd16.wait()
    plsc.subcore_barrier()

    cd = {}

    def count_scatter(k):
        cd[k] = pltpu.async_copy(
            ones_v, cnt_sh.at[dstall.at[pl.ds(k * CHUNK, CHUNK)]],
            csem, add=True)
        if k >= 2:
            cd[k - 2].wait()

    _agg_pipeline(table, srcall, dstall, rows_v, acc_sh, gsem, ssem,
                  per_chunk=count_scatter)
    cd[NCHUNK - 2].wait()
    cd[NCHUNK - 1].wait()
    plsc.subcore_barrier()
    pltpu.sync_copy(acc_sh.at[pl.ds(s * RPT, RPT)],
                    out.at[c, pl.ds(s * RPT, RPT)])
    pltpu.sync_copy(cnt_sh.at[pl.ds(s * RPT, RPT)],
                    cnt_out.at[c, pl.ds(s * RPT, RPT)])


_sc_agg0 = functools.partial(
    pl.kernel,
    out_type=(jax.ShapeDtypeStruct((NC, N_PAD, H), jnp.float32),
              jax.ShapeDtypeStruct((NC, N_PAD, 16), jnp.float32)),
    mesh=_MESH,
    compiler_params=_SC_PARAMS,
    scratch_types=[
        pltpu.VMEM((EPT,), jnp.int32),
        pltpu.VMEM((EPT,), jnp.int32),
        pltpu.VMEM((2, CHUNK, H), jnp.float32),
        pltpu.VMEM((CHUNK, 16), jnp.float32),
        pltpu.VMEM_SHARED((N_PAD, H), jnp.float32),
        pltpu.VMEM_SHARED((N_PAD, 16), jnp.float32),
        pltpu.SemaphoreType.DMA,
        pltpu.SemaphoreType.DMA,
        pltpu.SemaphoreType.DMA,
        pltpu.SemaphoreType.DMA,
    ],
)(_sc_agg0_body)


def _sc_agg_body(table, edges, zeros, out, srcall, dstall, rows_v, acc_sh,
                 zsem, gsem, ssem):
    c = lax.axis_index("c")
    s = lax.axis_index("s")
    wid = c * NS + s
    base = wid * EPT
    zd = pltpu.async_copy(zeros.at[pl.ds(s * RPT, RPT)],
                          acc_sh.at[pl.ds(s * RPT, RPT)], zsem)
    pltpu.sync_copy(edges.at[0, pl.ds(base, EPT)], srcall)
    pltpu.sync_copy(edges.at[1, pl.ds(base, EPT)], dstall)
    zd.wait()
    plsc.subcore_barrier()
    _agg_pipeline(table, srcall, dstall, rows_v, acc_sh, gsem, ssem)
    plsc.subcore_barrier()
    pltpu.sync_copy(acc_sh.at[pl.ds(s * RPT, RPT)],
                    out.at[c, pl.ds(s * RPT, RPT)])


_sc_agg = functools.partial(
    pl.kernel,
    out_type=jax.ShapeDtypeStruct((NC, N_PAD, H), jnp.float32),
    mesh=_MESH,
    compiler_params=_SC_PARAMS,
    scratch_types=[
        pltpu.VMEM((EPT,), jnp.int32),
        pltpu.VMEM((EPT,), jnp.int32),
        pltpu.VMEM((2, CHUNK, H), jnp.float32),
        pltpu.VMEM_SHARED((N_PAD, H), jnp.float32),
        pltpu.SemaphoreType.DMA,
        pltpu.SemaphoreType.DMA,
        pltpu.SemaphoreType.DMA,
    ],
)(_sc_agg_body)


def _pre_body(x_ref, wl_ref, wr_ref, hl_ref, hr_ref):
    x = x_ref[...]
    hl_ref[...] = jnp.dot(x, wl_ref[...], preferred_element_type=jnp.float32)
    hr_ref[...] = jnp.dot(x, wr_ref[...], preferred_element_type=jnp.float32)


_tc_pre = pl.pallas_call(
    _pre_body,
    out_shape=(jax.ShapeDtypeStruct((N, H), jnp.float32),
               jax.ShapeDtypeStruct((N, H), jnp.float32)),
)


def _norm_relu(acc_ref, cnt_ref, hr_ref, bl_ref, g_ref, b_ref):
    agg = acc_ref[0, :N, :] + acc_ref[1, :N, :]
    cnt = cnt_ref[0, :N, 0:1] + cnt_ref[1, :N, 0:1]
    t = agg / jnp.maximum(cnt, 1.0) + bl_ref[...] + hr_ref[...]
    mu = jnp.mean(t, axis=0, keepdims=True)
    var = jnp.mean((t - mu) * (t - mu), axis=0, keepdims=True)
    hn = (t - mu) / jnp.sqrt(var + 1e-5) * g_ref[...] + b_ref[...]
    return jnp.maximum(hn, 0.0)


def _mid_body(acc_ref, cnt_ref, hr_ref, bl_ref, g_ref, b_ref,
              wl_ref, wr_ref, hl_o, hr_o):
    h = _norm_relu(acc_ref, cnt_ref, hr_ref, bl_ref, g_ref, b_ref)
    hl_o[...] = jnp.dot(h, wl_ref[...], preferred_element_type=jnp.float32)
    hr_o[...] = jnp.dot(h, wr_ref[...], preferred_element_type=jnp.float32)


_tc_mid = pl.pallas_call(
    _mid_body,
    out_shape=(jax.ShapeDtypeStruct((N, H), jnp.float32),
               jax.ShapeDtypeStruct((N, H), jnp.float32)),
)


def _post_body(acc_ref, cnt_ref, hr_ref, bl_ref, g_ref, b_ref,
               batch_ref, gattr_ref, w1_ref, b1_ref, w2_ref, b2_ref,
               out_ref):
    h = _norm_relu(acc_ref, cnt_ref, hr_ref, bl_ref, g_ref, b_ref)
    gid = lax.broadcasted_iota(jnp.int32, (G, N), 0)
    mask = (gid == batch_ref[...]).astype(jnp.float32)
    s = jnp.dot(mask, h, preferred_element_type=jnp.float32)
    cntg = jnp.sum(mask, axis=1, keepdims=True)
    emb = s / jnp.maximum(cntg, 1.0)
    w1 = w1_ref[...]
    z = (jnp.dot(emb, w1[:H], preferred_element_type=jnp.float32)
         + jnp.dot(gattr_ref[...], w1[H:], preferred_element_type=jnp.float32)
         + b1_ref[...])
    z = jnp.maximum(z, 0.0)
    out_ref[...] = (jnp.dot(z, w2_ref[...], preferred_element_type=jnp.float32)
                    + b2_ref[...])


_tc_post = pl.pallas_call(
    _post_body,
    out_shape=jax.ShapeDtypeStruct((G, 2), jnp.float32),
)


def kernel(x, graph_attr, params, edge_index, batch):
    zeros64 = jnp.zeros((N_PAD, H), jnp.float32)
    zeros16 = jnp.zeros((N_PAD, 16), jnp.float32)
    ones16 = jnp.ones((CHUNK, 16), jnp.float32)

    p0 = params['conv0']
    hl, hr = _tc_pre(x, p0['Wl'], p0['Wr'])

    out = None
    cnt2 = None
    for i in range(3):
        p = params['conv%d' % i]
        if i == 0:
            acc2, cnt2 = _sc_agg0(hl, edge_index, zeros64, zeros16, ones16)
        else:
            acc2 = _sc_agg(hl, edge_index, zeros64)
        if i < 2:
            pn = params['conv%d' % (i + 1)]
            hl, hr = _tc_mid(acc2, cnt2, hr, p['bl'], p['gamma'], p['beta'],
                             pn['Wl'], pn['Wr'])
        else:
            out = _tc_post(acc2, cnt2, hr, p['bl'], p['gamma'], p['beta'],
                           batch, graph_attr,
                           params['W1'], params['b1'],
                           params['W2'], params['b2'])
    return out
